# baseline (speedup 1.0000x reference)
import jax
import jax.numpy as jnp
from jax import lax
from jax.experimental import pallas as pl
from jax.experimental.pallas import tpu as pltpu
from jax.experimental.pallas import tpu_sc as plsc

N = 10000
D = 128
SAMPLE = 3

BR = 1024
BK = 2048
KB = 5
NPAD = 10240
HALF = 5120
RBH = 5
N2 = N - HALF

BC = 512
CBH = 10

NW = 32
BPW = 480
ICH = 120
NCH = 4

_LOWM = 0x3FFF
_HIGHM = ~_LOWM


def _fused_body(adjl_ref, adjr_ref, x_ref, andm_ref,
                orm_ref, hi_ref, idx_ref, acc_ref, rk_ref):
    k = pl.program_id(1)
    nk = pl.num_programs(1)

    @pl.when(k == 0)
    def _init():
        acc_ref[...] = jnp.zeros_like(acc_ref)
        rk_ref[...] = jnp.full_like(rk_ref, -jnp.inf)

    h = BK // 2
    andm = andm_ref[...]
    orm = orm_ref[...]

    a0l = jnp.where(andm[:, :h] < 0, adjl_ref[...], 0.0)
    a0r = jnp.where(andm[:, h:] < 0, adjr_ref[...], 0.0)
    acc_ref[...] += (
        jnp.dot(a0l, x_ref[:h, :], preferred_element_type=jnp.float32)
        + jnp.dot(a0r, x_ref[h:, :], preferred_element_type=jnp.float32))

    ms = []
    for aref, sl in ((adjl_ref, 0), (adjr_ref, h)):
        bits = lax.bitcast_convert_type(aref[...], jnp.int32)
        key_i = (bits & andm[:, sl:sl + h]) | orm[:, sl:sl + h]
        key = lax.bitcast_convert_type(key_i, jnp.float32)
        for t in range(SAMPLE):
            m = jnp.max(key, axis=1, keepdims=True)
            ms.append(m)
            if t < SAMPLE - 1:
                key = jnp.where(key == m, -jnp.inf, key)

    pos = lax.broadcasted_iota(jnp.int32, (BR, 32), 1)
    base = 6 * k
    parked = rk_ref[...]
    for i in range(6):
        parked = jnp.where(pos == base + i, ms[i], parked)
    rk_ref[...] = parked

    @pl.when(k == nk - 1)
    def _fin():
        hi_ref[...] = acc_ref[...]
        cand = rk_ref[...]
        picks = []
        for t in range(SAMPLE):
            m = jnp.max(cand, axis=1, keepdims=True)
            mi = lax.bitcast_convert_type(m, jnp.int32)
            picks.append((mi & _LOWM) ^ _LOWM)
            if t < SAMPLE - 1:
                cand = jnp.where(cand == m, -jnp.inf, cand)
        ipad5 = jnp.full((BR, 5), 0, jnp.int32)
        idx_ref[...] = jnp.concatenate(picks + [ipad5], axis=1)


def _fused_call(adj, x_pad, andm, orm, roff, nrows, interpret=False):
    return pl.pallas_call(
        _fused_body,
        grid=(RBH, KB),
        in_specs=[
            pl.BlockSpec((BR, BK // 2), lambda r, k: (r + roff, 2 * k)),
            pl.BlockSpec((BR, BK // 2), lambda r, k: (r + roff, 2 * k + 1)),
            pl.BlockSpec((BK, D), lambda r, k: (k, 0)),
            pl.BlockSpec((1, BK), lambda r, k: (0, k)),
            pl.BlockSpec((1, BK), lambda r, k: (0, k)),
        ],
        out_specs=[
            pl.BlockSpec((BR, D), lambda r, k: (r, 0)),
            pl.BlockSpec((BR, 8), lambda r, k: (r, 0)),
        ],
        out_shape=[
            jax.ShapeDtypeStruct((nrows, D), jnp.float32),
            jax.ShapeDtypeStruct((nrows, 8), jnp.int32),
        ],
        scratch_shapes=[
            pltpu.VMEM((BR, D), jnp.float32),
            pltpu.VMEM((BR, 32), jnp.float32),
        ],
        compiler_params=pltpu.CompilerParams(
            dimension_semantics=("parallel", "arbitrary")),
        interpret=interpret,
    )(adj, adj, x_pad, andm, orm)


def _sc_gather_body(idx_hbm, table_hbm, out_hbm, idx_v, rows_v, sem):
    wid = lax.axis_index("s") * 2 + lax.axis_index("c")
    pltpu.sync_copy(idx_hbm.at[wid], idx_v)
    copies = [
        pltpu.async_copy(
            table_hbm.at[idx_v.at[j]],
            rows_v.at[pl.ds(j * ICH, ICH)],
            sem,
        )
        for j in range(NCH)
    ]
    for c in copies:
        c.wait()
    pltpu.sync_copy(rows_v, out_hbm.at[pl.ds(wid * BPW, BPW)])


def _sc_gather_call(idx_chunks, table):
    return pl.kernel(
        _sc_gather_body,
        out_type=jax.ShapeDtypeStruct((NW * BPW, D), jnp.float32),
        mesh=plsc.VectorSubcoreMesh(core_axis_name="c", subcore_axis_name="s"),
        scratch_types=[
            pltpu.VMEM((NCH, ICH), jnp.int32),
            pltpu.VMEM((BPW, D), jnp.float32),
            pltpu.SemaphoreType.DMA,
        ],
    )(idx_chunks, table)


def _epilogue_body(coef_ref, hi_ref, x_ref, h0_ref, g0_ref, g1_ref, g2_ref,
                   w_ref, o_ref):
    w = w_ref[...]
    mm = jnp.dot(hi_ref[...], w[:D, :], preferred_element_type=jnp.float32)
    mm += jnp.dot(x_ref[...], w[D:, :], preferred_element_type=jnp.float32)
    gsum = g0_ref[...] + g1_ref[...] + g2_ref[...]
    o_ref[...] = (coef_ref[0] * mm + coef_ref[1] * x_ref[...]
                  + coef_ref[2] * h0_ref[...] + coef_ref[3] * gsum)


def _epilogue_body_alias(coef_ref, hi_ref, x_ref, h0_ref, g0_ref, g1_ref,
                         g2_ref, w_ref, prev_ref, o_ref):
    _epilogue_body(coef_ref, hi_ref, x_ref, h0_ref, g0_ref, g1_ref, g2_ref,
                   w_ref, o_ref)


def _epilogue_call(coefs, hi, x, h0, g, w, roff, prev=None, interpret=False):
    half_spec = pl.BlockSpec((BC, D), lambda r: (r + roff, 0))
    in_specs = [
        pl.BlockSpec(memory_space=pltpu.SMEM),
        pl.BlockSpec((BC, D), lambda r: (r, 0)),
        half_spec,
        half_spec,
        pl.BlockSpec((BC, D), lambda r: (r, 0)),
        pl.BlockSpec((BC, D), lambda r: (r + CBH, 0)),
        pl.BlockSpec((BC, D), lambda r: (r + 2 * CBH, 0)),
        pl.BlockSpec((2 * D, D), lambda r: (0, 0)),
    ]
    args = [coefs, hi, x, h0, g, g, g, w]
    body = _epilogue_body
    aliases = {}
    if prev is not None:
        in_specs.append(pl.BlockSpec((8, D), lambda r: (0, 0)))
        args.append(prev)
        body = _epilogue_body_alias
        aliases = {8: 0}
    return pl.pallas_call(
        body,
        grid=(CBH,),
        in_specs=in_specs,
        out_specs=pl.BlockSpec((BC, D), lambda r: (r + roff, 0)),
        out_shape=jax.ShapeDtypeStruct((N, D), jnp.float32),
        input_output_aliases=aliases,
        compiler_params=pltpu.CompilerParams(
            dimension_semantics=("arbitrary",)),
        interpret=interpret,
    )(*args)


def _idx_chunks(idx8, nrows):
    idx3 = idx8[:, :SAMPLE]
    idx_flat = jnp.pad(idx3, ((0, HALF - nrows), (0, 0))).T
    return idx_flat.reshape(NW, NCH, ICH)


def kernel(input, adj, h0, W, lamda, alpha, l):
    x = input
    theta = jnp.minimum(1.0, jnp.log(lamda / l + 1.0)).astype(jnp.float32)
    alpha = jnp.asarray(alpha, jnp.float32)
    coefs = jnp.stack([
        theta,
        (1.0 - theta) * (1.0 - alpha),
        (1.0 - theta) * alpha,
        (1.0 - theta) * 0.1 / 3.0,
    ]).astype(jnp.float32)

    x_pad = jnp.pad(x, ((0, NPAD - N), (0, 0)))
    colg = jnp.arange(NPAD, dtype=jnp.int32)
    valid = colg < N
    andm = jnp.where(valid, _HIGHM, 0).astype(jnp.int32).reshape(1, NPAD)
    orm = jnp.where(valid, colg ^ _LOWM, 0).astype(jnp.int32).reshape(1, NPAD)

    hi1, idx81 = _fused_call(adj, x_pad, andm, orm, 0, HALF)
    g1 = _sc_gather_call(_idx_chunks(idx81, HALF), x)
    hi2, idx82 = _fused_call(adj, x_pad, andm, orm, RBH, N2)
    g2 = _sc_gather_call(_idx_chunks(idx82, N2), x)

    out1 = _epilogue_call(coefs, hi1, x, h0, g1, W, 0)
    return _epilogue_call(coefs, hi2, x, h0, g2, W, CBH, prev=out1)


# speedup vs baseline: 1.0087x; 1.0087x over previous
import functools

import jax
import jax.numpy as jnp
from jax import lax
from jax.experimental import pallas as pl
from jax.experimental.pallas import tpu as pltpu
from jax.experimental.pallas import tpu_sc as plsc

N = 10000
D = 128
SAMPLE = 3

BR = 1024
BK = 2048
RB = 10
KB = 5
NPAD = 10240

BC = 512
CB = 20

NW = 32
BPW = 960
ICH = 120
NCH = 8

_BIGI = 2 ** 30


_LOWM = 0x3FFF
_HIGHM = ~_LOWM
_IMIN = -2 ** 31
_IMAX = 2 ** 31 - 1


def _fused_body(adjl_ref, adjr_ref, x_ref, inv_ref,
                cap_ref, hi_ref, idx_ref, acc_ref, rk_ref):
    k = pl.program_id(1)
    nk = pl.num_programs(1)

    @pl.when(k == 0)
    def _init():
        acc_ref[...] = jnp.zeros_like(acc_ref)
        rk_ref[...] = jnp.full_like(rk_ref, -jnp.inf)

    h = BK // 2
    andm = inv_ref[...]
    orm = cap_ref[...]

    a0l = jnp.where(andm[:, :h] < 0, adjl_ref[...], 0.0)
    a0r = jnp.where(andm[:, h:] < 0, adjr_ref[...], 0.0)
    acc_ref[...] += (
        jnp.dot(a0l, x_ref[:h, :], preferred_element_type=jnp.float32)
        + jnp.dot(a0r, x_ref[h:, :], preferred_element_type=jnp.float32))

    ms = []
    for aref, sl in ((adjl_ref, 0), (adjr_ref, h)):
        bits = lax.bitcast_convert_type(aref[...], jnp.int32)
        key_i = (bits & andm[:, sl:sl + h]) | orm[:, sl:sl + h]
        key = lax.bitcast_convert_type(key_i, jnp.float32)
        for t in range(SAMPLE):
            m = jnp.max(key, axis=1, keepdims=True)
            ms.append(m)
            if t < SAMPLE - 1:
                key = jnp.where(key == m, -jnp.inf, key)

    pos = lax.broadcasted_iota(jnp.int32, (BR, 32), 1)
    base = 6 * k
    parked = rk_ref[...]
    for i in range(6):
        parked = jnp.where(pos == base + i, ms[i], parked)
    rk_ref[...] = parked

    @pl.when(k == nk - 1)
    def _fin():
        hi_ref[...] = acc_ref[...]
        cand = rk_ref[...]
        picks = []
        for t in range(SAMPLE):
            m = jnp.max(cand, axis=1, keepdims=True)
            mi = lax.bitcast_convert_type(m, jnp.int32)
            picks.append((mi & _LOWM) ^ _LOWM)
            if t < SAMPLE - 1:
                cand = jnp.where(cand == m, -jnp.inf, cand)
        ipad5 = jnp.full((BR, 5), 0, jnp.int32)
        idx_ref[...] = jnp.concatenate(picks + [ipad5], axis=1)

def _fused_call(adj, x_pad, inv, cap, interpret=False):
    return pl.pallas_call(
        _fused_body,
        grid=(RB, KB),
        in_specs=[
            pl.BlockSpec((BR, BK // 2), lambda r, k: (r, 2 * k)),
            pl.BlockSpec((BR, BK // 2), lambda r, k: (r, 2 * k + 1)),
            pl.BlockSpec((BK, D), lambda r, k: (k, 0)),
            pl.BlockSpec((1, BK), lambda r, k: (0, k)),
            pl.BlockSpec((1, BK), lambda r, k: (0, k)),
        ],
        out_specs=[
            pl.BlockSpec((BR, D), lambda r, k: (r, 0)),
            pl.BlockSpec((BR, 8), lambda r, k: (r, 0)),
        ],
        out_shape=[
            jax.ShapeDtypeStruct((N, D), jnp.float32),
            jax.ShapeDtypeStruct((N, 8), jnp.int32),
        ],
        scratch_shapes=[
            pltpu.VMEM((BR, D), jnp.float32),
            pltpu.VMEM((BR, 32), jnp.float32),
        ],
        compiler_params=pltpu.CompilerParams(
            dimension_semantics=("parallel", "arbitrary")),
        interpret=interpret,
    )(adj, adj, x_pad, inv, cap)


def _sc_gather_body(idx_hbm, table_hbm, out_hbm, idx_v, rows_v, sem):
    wid = lax.axis_index("s") * 2 + lax.axis_index("c")
    pltpu.sync_copy(idx_hbm.at[wid], idx_v)
    copies = [
        pltpu.async_copy(
            table_hbm.at[idx_v.at[j]],
            rows_v.at[pl.ds(j * ICH, ICH)],
            sem,
        )
        for j in range(NCH)
    ]
    for c in copies:
        c.wait()
    pltpu.sync_copy(rows_v, out_hbm.at[pl.ds(wid * BPW, BPW)])


def _sc_gather_call(idx_chunks, table):
    return pl.kernel(
        _sc_gather_body,
        out_type=jax.ShapeDtypeStruct((NW * BPW, D), jnp.float32),
        mesh=plsc.VectorSubcoreMesh(core_axis_name="c", subcore_axis_name="s"),
        scratch_types=[
            pltpu.VMEM((NCH, ICH), jnp.int32),
            pltpu.VMEM((BPW, D), jnp.float32),
            pltpu.SemaphoreType.DMA,
        ],
    )(idx_chunks, table)


def _epilogue_body(coef_ref, hi_ref, x_ref, h0_ref, g0_ref, g1_ref, g2_ref,
                   w_ref, o_ref):
    w = w_ref[...]
    mm = jnp.dot(hi_ref[...], w[:D, :], preferred_element_type=jnp.float32)
    mm += jnp.dot(x_ref[...], w[D:, :], preferred_element_type=jnp.float32)
    gsum = g0_ref[...] + g1_ref[...] + g2_ref[...]
    o_ref[...] = (coef_ref[0] * mm + coef_ref[1] * x_ref[...]
                  + coef_ref[2] * h0_ref[...] + coef_ref[3] * gsum)


def _epilogue_call(coefs, hi, x, h0, g, w, interpret=False):
    row_spec = pl.BlockSpec((BC, D), lambda r: (r, 0))
    return pl.pallas_call(
        _epilogue_body,
        grid=(CB,),
        in_specs=[
            pl.BlockSpec(memory_space=pltpu.SMEM),
            row_spec,
            row_spec,
            row_spec,
            pl.BlockSpec((BC, D), lambda r: (r, 0)),
            pl.BlockSpec((BC, D), lambda r: (r + CB, 0)),
            pl.BlockSpec((BC, D), lambda r: (r + 2 * CB, 0)),
            pl.BlockSpec((2 * D, D), lambda r: (0, 0)),
        ],
        out_specs=pl.BlockSpec((BC, D), lambda r: (r, 0)),
        out_shape=jax.ShapeDtypeStruct((N, D), jnp.float32),
        compiler_params=pltpu.CompilerParams(
            dimension_semantics=("arbitrary",)),
        interpret=interpret,
    )(coefs, hi, x, h0, g, g, g, w)


def kernel(input, adj, h0, W, lamda, alpha, l):
    x = input
    theta = jnp.minimum(1.0, jnp.log(lamda / l + 1.0)).astype(jnp.float32)
    alpha = jnp.asarray(alpha, jnp.float32)
    coefs = jnp.stack([
        theta,
        (1.0 - theta) * (1.0 - alpha),
        (1.0 - theta) * alpha,
        (1.0 - theta) * 0.1 / 3.0,
    ]).astype(jnp.float32)

    x_pad = jnp.pad(x, ((0, NPAD - N), (0, 0)))
    colg = jnp.arange(NPAD, dtype=jnp.int32)
    valid = colg < N
    andm = jnp.where(valid, _HIGHM, 0).astype(jnp.int32).reshape(1, NPAD)
    orm = jnp.where(valid, colg ^ _LOWM, 0).astype(jnp.int32).reshape(1, NPAD)
    hi, idx8 = _fused_call(adj, x_pad, andm, orm)

    idx3 = idx8[:, :SAMPLE]
    idx_flat = jnp.pad(idx3, ((0, NPAD - N), (0, 0))).T
    idx_chunks = idx_flat.reshape(NW, NCH, ICH)
    g = _sc_gather_call(idx_chunks, x)

    return _epilogue_call(coefs, hi, x, h0, g, W)
